# baseline (speedup 1.0000x reference)
import functools

import jax
import jax.numpy as jnp
from jax import lax
from jax.experimental import pallas as pl
from jax.experimental.pallas import tpu as pltpu
from jax.experimental.pallas import tpu_sc as plsc

KNN = 3
B = 4
N = 16384
M = 4096
D = 256

TN = 512
NT = N // TN

NC = 2
NS = 16
LANES = 16
NW = NC * NS
PW = N // NW
C = 32
NCHUNK = PW // C


def _topk_body(xyz_ref, sxyz_ref, mask_ref, gidx_ref, wb_ref):
    x = xyz_ref[...]
    s = sxyz_ref[...]
    mask = mask_ref[...]

    x16 = x.astype(jnp.bfloat16)
    s16 = s.astype(jnp.bfloat16)
    a2 = jnp.sum(x * x, axis=1, keepdims=True)
    b2 = jnp.sum(s * s, axis=0, keepdims=True)
    cross = lax.dot_general(x16, s16, (((1,), (0,)), ((), ())),
                            preferred_element_type=jnp.float32)
    d = a2 + b2 - 2.0 * cross

    iota = lax.broadcasted_iota(jnp.int32, (TN, M), 1).astype(jnp.float32)
    inf = jnp.float32(jnp.inf)
    mf = jnp.float32(M)

    def extract(dcur):
        mval = jnp.min(dcur, axis=1, keepdims=True)
        aidx = jnp.min(jnp.where(dcur == mval, iota, mf), axis=1, keepdims=True)
        return mval, aidx

    m1, a1 = extract(d)
    d = jnp.where(iota == a1, inf, d)
    m2, a2 = extract(d)
    d = jnp.where(iota == a2, inf, d)
    m3, a3 = extract(d)

    def weight(mv):
        v = jnp.maximum(mv, jnp.float32(1e-10))
        return 1.0 / (v * v + jnp.float32(1e-10))

    w1 = weight(m1)
    w2 = weight(m2)
    w3 = weight(m3)
    wsum = w1 + w2 + w3
    w1 = w1 / wsum * mask
    w2 = w2 / wsum * mask
    w3 = w3 / wsum * mask

    gidx_ref[...] = jnp.concatenate([a1, a2, a3], axis=1).astype(jnp.int32)
    wb_ref[...] = jnp.concatenate([
        jnp.broadcast_to(w1, (TN, LANES)),
        jnp.broadcast_to(w2, (TN, LANES)),
        jnp.broadcast_to(w3, (TN, LANES)),
    ], axis=1)


def _tc_topk(xyz_b, sxyz_t_b, mask_b):
    R = xyz_b.shape[0]
    return pl.pallas_call(
        _topk_body,
        grid=(R // TN,),
        in_specs=[
            pl.BlockSpec((TN, 3), lambda t: (t, 0)),
            pl.BlockSpec((3, M), lambda t: (0, 0)),
            pl.BlockSpec((TN, 1), lambda t: (t, 0)),
        ],
        out_specs=[
            pl.BlockSpec((TN, KNN), lambda t: (t, 0)),
            pl.BlockSpec((TN, KNN * LANES), lambda t: (t, 0)),
        ],
        out_shape=[
            jax.ShapeDtypeStruct((R, KNN), jnp.int32),
            jax.ShapeDtypeStruct((R, KNN * LANES), jnp.float32),
        ],
    )(xyz_b, sxyz_t_b, mask_b)


def _sc_interp(table, gidx, wb):
    mesh = plsc.VectorSubcoreMesh(core_axis_name="c", subcore_axis_name="s")
    G = KNN * C
    R = gidx.shape[0] // KNN
    PW = R // NW
    NCHUNK = PW // C

    @functools.partial(
        pl.kernel,
        out_type=jax.ShapeDtypeStruct((R, D), jnp.float32),
        mesh=mesh,
        scratch_types=[
            pltpu.VMEM((G,), jnp.int32),
            pltpu.VMEM((G,), jnp.int32),
            pltpu.VMEM((G, LANES), jnp.float32),
            pltpu.VMEM((G, LANES), jnp.float32),
            pltpu.VMEM((G, D), jnp.float32),
            pltpu.VMEM((G, D), jnp.float32),
            pltpu.VMEM((C, D), jnp.float32),
            pltpu.SemaphoreType.DMA,
            pltpu.SemaphoreType.DMA,
            pltpu.SemaphoreType.DMA,
            pltpu.SemaphoreType.DMA,
            pltpu.SemaphoreType.DMA,
            pltpu.SemaphoreType.DMA,
        ],
    )
    def sck(table_hbm, gidx_hbm, wb_hbm, out_hbm,
            idx0, idx1, w0, w1, rows0, rows1, out_v,
            si0, si1, sw0, sw1, sg0, sg1):
        wid = lax.axis_index("s") * NC + lax.axis_index("c")
        idx_v = (idx0, idx1)
        w_v = (w0, w1)
        rows_v = (rows0, rows1)
        si = (si0, si1)
        sw = (sw0, sw1)
        sg = (sg0, sg1)

        def start_idx(ch, slot):
            @pl.when(ch < NCHUNK)
            def _():
                ibase = (wid * PW + ch * C) * KNN
                pltpu.async_copy(gidx_hbm.at[pl.ds(ibase, G)], idx_v[slot],
                                 si[slot])

        def start_w(ch, slot):
            @pl.when(ch < NCHUNK)
            def _():
                ibase = (wid * PW + ch * C) * KNN
                pltpu.async_copy(wb_hbm.at[pl.ds(ibase, G)], w_v[slot],
                                 sw[slot])

        def wait_idx(slot):
            pltpu.make_async_copy(gidx_hbm.at[pl.ds(0, G)], idx_v[slot],
                                  si[slot]).wait()

        def start_gather(slot):
            pltpu.async_copy(table_hbm.at[idx_v[slot]], rows_v[slot], sg[slot])

        def wait_gather(slot):
            pltpu.make_async_copy(table_hbm.at[idx_v[slot]], rows_v[slot],
                                  sg[slot]).wait()
            pltpu.make_async_copy(wb_hbm.at[pl.ds(0, G)], w_v[slot],
                                  sw[slot]).wait()

        def compute(ch, slot):
            rv = rows_v[slot]
            wv = w_v[slot]

            @pl.loop(0, C)
            def _(r):
                a = wv[KNN * r]
                b = wv[KNN * r + 1]
                c_ = wv[KNN * r + 2]
                for c in range(D // LANES):
                    sl = pl.ds(c * LANES, LANES)
                    out_v[r, sl] = (rv[KNN * r, sl] * a +
                                    rv[KNN * r + 1, sl] * b +
                                    rv[KNN * r + 2, sl] * c_)

            pltpu.sync_copy(out_v, out_hbm.at[pl.ds(wid * PW + ch * C, C)])

        start_idx(0, 0)
        start_w(0, 0)
        wait_idx(0)
        start_gather(0)
        start_idx(1, 1)
        start_w(1, 1)

        @pl.loop(0, NCHUNK, step=2)
        def _(ch):
            wait_idx(1)
            start_gather(1)
            wait_gather(0)
            start_idx(ch + 2, 0)
            compute(ch, 0)
            start_w(ch + 2, 0)

            @pl.when(ch + 2 < NCHUNK)
            def _():
                wait_idx(0)
                start_gather(0)

            wait_gather(1)
            start_idx(ch + 3, 1)
            compute(ch + 1, 1)
            start_w(ch + 3, 1)

    return sck(table, gidx, wb)


@jax.jit
def kernel(xyz, sampled_xyz, features, sampled_features, masks):
    del features
    sxyz_t = sampled_xyz.transpose(0, 2, 1)
    mask_f = masks.astype(jnp.float32).reshape(B, N, 1)
    H = N // 2
    outs = []
    for b in range(B):
        halves = []
        for h in range(2):
            sl = slice(h * H, (h + 1) * H)
            gidx, wb = _tc_topk(xyz[b, sl], sxyz_t[b], mask_f[b, sl])
            halves.append(_sc_interp(sampled_features[b],
                                     gidx.reshape(H * KNN),
                                     wb.reshape(H * KNN, LANES)))
        outs.append(jnp.concatenate(halves, axis=0))
    return jnp.stack(outs, axis=0)


# speedup vs baseline: 1.0197x; 1.0197x over previous
import functools

import jax
import jax.numpy as jnp
from jax import lax
from jax.experimental import pallas as pl
from jax.experimental.pallas import tpu as pltpu
from jax.experimental.pallas import tpu_sc as plsc

KNN = 3
B = 4
N = 16384
M = 4096
D = 256

TN = 512
NT = N // TN

NC = 2
NS = 16
LANES = 16
NW = NC * NS
PW = N // NW
C = 32
NCHUNK = PW // C


def _topk_body(xyz_ref, sxyz_ref, mask_ref, gidx_ref, wb_ref):
    x = xyz_ref[...]
    s = sxyz_ref[...]
    mask = mask_ref[...]

    x16 = x.astype(jnp.bfloat16)
    s16 = s.astype(jnp.bfloat16)
    a2 = jnp.sum(x * x, axis=1, keepdims=True)
    b2 = jnp.sum(s * s, axis=0, keepdims=True)
    cross = lax.dot_general(x16, s16, (((1,), (0,)), ((), ())),
                            preferred_element_type=jnp.float32)
    d = a2 + b2 - 2.0 * cross

    iota = lax.broadcasted_iota(jnp.int32, (TN, M), 1).astype(jnp.float32)
    inf = jnp.float32(jnp.inf)
    mf = jnp.float32(M)

    def extract(dcur):
        mval = jnp.min(dcur, axis=1, keepdims=True)
        aidx = jnp.min(jnp.where(dcur == mval, iota, mf), axis=1, keepdims=True)
        return mval, aidx

    m1, a1 = extract(d)
    d = jnp.where(iota == a1, inf, d)
    m2, a2 = extract(d)
    d = jnp.where(iota == a2, inf, d)
    m3, a3 = extract(d)

    def weight(mv):
        v = jnp.maximum(mv, jnp.float32(1e-10))
        return 1.0 / (v * v + jnp.float32(1e-10))

    w1 = weight(m1)
    w2 = weight(m2)
    w3 = weight(m3)
    wsum = w1 + w2 + w3
    w1 = w1 / wsum * mask
    w2 = w2 / wsum * mask
    w3 = w3 / wsum * mask

    gidx_ref[...] = jnp.concatenate([a1, a2, a3], axis=1).astype(jnp.int32)
    wb_ref[...] = jnp.concatenate([
        jnp.broadcast_to(w1, (TN, LANES)),
        jnp.broadcast_to(w2, (TN, LANES)),
        jnp.broadcast_to(w3, (TN, LANES)),
    ], axis=1)


def _tc_topk(xyz_b, sxyz_t_b, mask_b):
    R = xyz_b.shape[0]
    return pl.pallas_call(
        _topk_body,
        grid=(R // TN,),
        in_specs=[
            pl.BlockSpec((TN, 3), lambda t: (t, 0)),
            pl.BlockSpec((3, M), lambda t: (0, 0)),
            pl.BlockSpec((TN, 1), lambda t: (t, 0)),
        ],
        out_specs=[
            pl.BlockSpec((TN, KNN), lambda t: (t, 0)),
            pl.BlockSpec((TN, KNN * LANES), lambda t: (t, 0)),
        ],
        out_shape=[
            jax.ShapeDtypeStruct((R, KNN), jnp.int32),
            jax.ShapeDtypeStruct((R, KNN * LANES), jnp.float32),
        ],
    )(xyz_b, sxyz_t_b, mask_b)


def _sc_interp(table, gidx, wb):
    mesh = plsc.VectorSubcoreMesh(core_axis_name="c", subcore_axis_name="s")
    G = KNN * C
    R = gidx.shape[0] // KNN
    PW = R // NW
    NCHUNK = PW // C

    @functools.partial(
        pl.kernel,
        out_type=jax.ShapeDtypeStruct((R, D), jnp.float32),
        mesh=mesh,
        scratch_types=[
            pltpu.VMEM((G,), jnp.int32),
            pltpu.VMEM((G,), jnp.int32),
            pltpu.VMEM((G, LANES), jnp.float32),
            pltpu.VMEM((G, LANES), jnp.float32),
            pltpu.VMEM((G, D), jnp.float32),
            pltpu.VMEM((G, D), jnp.float32),
            pltpu.VMEM((C, D), jnp.float32),
            pltpu.SemaphoreType.DMA,
            pltpu.SemaphoreType.DMA,
            pltpu.SemaphoreType.DMA,
            pltpu.SemaphoreType.DMA,
            pltpu.SemaphoreType.DMA,
            pltpu.SemaphoreType.DMA,
        ],
    )
    def sck(table_hbm, gidx_hbm, wb_hbm, out_hbm,
            idx0, idx1, w0, w1, rows0, rows1, out_v,
            si0, si1, sw0, sw1, sg0, sg1):
        wid = lax.axis_index("s") * NC + lax.axis_index("c")
        idx_v = (idx0, idx1)
        w_v = (w0, w1)
        rows_v = (rows0, rows1)
        si = (si0, si1)
        sw = (sw0, sw1)
        sg = (sg0, sg1)

        def start_idx(ch, slot):
            @pl.when(ch < NCHUNK)
            def _():
                ibase = (wid * PW + ch * C) * KNN
                pltpu.async_copy(gidx_hbm.at[pl.ds(ibase, G)], idx_v[slot],
                                 si[slot])

        def start_w(ch, slot):
            @pl.when(ch < NCHUNK)
            def _():
                ibase = (wid * PW + ch * C) * KNN
                pltpu.async_copy(wb_hbm.at[pl.ds(ibase, G)], w_v[slot],
                                 sw[slot])

        def wait_idx(slot):
            pltpu.make_async_copy(gidx_hbm.at[pl.ds(0, G)], idx_v[slot],
                                  si[slot]).wait()

        def start_gather(slot):
            pltpu.async_copy(table_hbm.at[idx_v[slot]], rows_v[slot], sg[slot])

        def wait_gather(slot):
            pltpu.make_async_copy(table_hbm.at[idx_v[slot]], rows_v[slot],
                                  sg[slot]).wait()
            pltpu.make_async_copy(wb_hbm.at[pl.ds(0, G)], w_v[slot],
                                  sw[slot]).wait()

        def compute(ch, slot):
            rv = rows_v[slot]
            wv = w_v[slot]

            @pl.loop(0, C)
            def _(r):
                a = wv[KNN * r]
                b = wv[KNN * r + 1]
                c_ = wv[KNN * r + 2]
                for c in range(D // LANES):
                    sl = pl.ds(c * LANES, LANES)
                    out_v[r, sl] = (rv[KNN * r, sl] * a +
                                    rv[KNN * r + 1, sl] * b +
                                    rv[KNN * r + 2, sl] * c_)

            pltpu.sync_copy(out_v, out_hbm.at[pl.ds(wid * PW + ch * C, C)])

        start_idx(0, 0)
        start_w(0, 0)
        wait_idx(0)
        start_gather(0)
        start_idx(1, 1)
        start_w(1, 1)

        @pl.loop(0, NCHUNK, step=2)
        def _(ch):
            wait_idx(1)
            start_gather(1)
            wait_gather(0)
            start_idx(ch + 2, 0)
            compute(ch, 0)
            start_w(ch + 2, 0)

            @pl.when(ch + 2 < NCHUNK)
            def _():
                wait_idx(0)
                start_gather(0)

            wait_gather(1)
            start_idx(ch + 3, 1)
            compute(ch + 1, 1)
            start_w(ch + 3, 1)

    return sck(table, gidx, wb)


@jax.jit
def kernel(xyz, sampled_xyz, features, sampled_features, masks):
    del features
    sxyz_t = sampled_xyz.transpose(0, 2, 1)
    mask_f = masks.astype(jnp.float32).reshape(B, N, 1)
    def piece(b, sl, rows):
        gidx, wb = _tc_topk(xyz[b, sl], sxyz_t[b], mask_f[b, sl])
        return _sc_interp(sampled_features[b],
                          gidx.reshape(rows * KNN),
                          wb.reshape(rows * KNN, LANES))

    outs = []
    for b in range(B - 1):
        outs.append(piece(b, slice(None), N))
    H = N // 2
    outs.append(jnp.concatenate(
        [piece(B - 1, slice(0, H), H), piece(B - 1, slice(H, N), H)], axis=0))
    return jnp.stack(outs, axis=0)


# speedup vs baseline: 1.0329x; 1.0129x over previous
import functools

import jax
import jax.numpy as jnp
from jax import lax
from jax.experimental import pallas as pl
from jax.experimental.pallas import tpu as pltpu
from jax.experimental.pallas import tpu_sc as plsc

KNN = 3
B = 4
N = 16384
M = 4096
D = 256

TN = 512
NT = N // TN

NC = 2
NS = 16
LANES = 16
NW = NC * NS
PW = N // NW
C = 32
NCHUNK = PW // C


def _topk_body(xyz_ref, sxyz_ref, mask_ref, gidx_ref, wb_ref):
    x = xyz_ref[...]
    s = sxyz_ref[...]
    mask = mask_ref[...]

    x16 = x.astype(jnp.bfloat16)
    s16 = s.astype(jnp.bfloat16)
    a2 = jnp.sum(x * x, axis=1, keepdims=True)
    b2 = jnp.sum(s * s, axis=0, keepdims=True)
    cross = lax.dot_general(x16, s16, (((1,), (0,)), ((), ())),
                            preferred_element_type=jnp.float32)
    d = a2 + b2 - 2.0 * cross

    iota = lax.broadcasted_iota(jnp.int32, (TN, M), 1).astype(jnp.float32)
    inf = jnp.float32(jnp.inf)
    mf = jnp.float32(M)

    def extract(dcur):
        mval = jnp.min(dcur, axis=1, keepdims=True)
        aidx = jnp.min(jnp.where(dcur == mval, iota, mf), axis=1, keepdims=True)
        return mval, aidx

    m1, a1 = extract(d)
    d = jnp.where(iota == a1, inf, d)
    m2, a2 = extract(d)
    d = jnp.where(iota == a2, inf, d)
    m3, a3 = extract(d)

    def weight(mv):
        v = jnp.maximum(mv, jnp.float32(1e-10))
        return 1.0 / (v * v + jnp.float32(1e-10))

    w1 = weight(m1)
    w2 = weight(m2)
    w3 = weight(m3)
    wsum = w1 + w2 + w3
    w1 = w1 / wsum * mask
    w2 = w2 / wsum * mask
    w3 = w3 / wsum * mask

    gidx_ref[...] = jnp.concatenate([a1, a2, a3], axis=1).astype(jnp.int32)
    wb_ref[...] = jnp.concatenate([
        jnp.broadcast_to(w1, (TN, LANES)),
        jnp.broadcast_to(w2, (TN, LANES)),
        jnp.broadcast_to(w3, (TN, LANES)),
    ], axis=1)


def _tc_topk(xyz_b, sxyz_t_b, mask_b):
    R = xyz_b.shape[0]
    return pl.pallas_call(
        _topk_body,
        grid=(R // TN,),
        in_specs=[
            pl.BlockSpec((TN, 3), lambda t: (t, 0)),
            pl.BlockSpec((3, M), lambda t: (0, 0)),
            pl.BlockSpec((TN, 1), lambda t: (t, 0)),
        ],
        out_specs=[
            pl.BlockSpec((TN, KNN), lambda t: (t, 0)),
            pl.BlockSpec((TN, KNN * LANES), lambda t: (t, 0)),
        ],
        out_shape=[
            jax.ShapeDtypeStruct((R, KNN), jnp.int32),
            jax.ShapeDtypeStruct((R, KNN * LANES), jnp.float32),
        ],
    )(xyz_b, sxyz_t_b, mask_b)


def _sc_interp(table, gidx, wb):
    mesh = plsc.VectorSubcoreMesh(core_axis_name="c", subcore_axis_name="s")
    G = KNN * C
    R = gidx.shape[0] // KNN
    PW = R // NW
    NCHUNK = PW // C

    @functools.partial(
        pl.kernel,
        out_type=jax.ShapeDtypeStruct((R, D), jnp.float32),
        mesh=mesh,
        scratch_types=[
            pltpu.VMEM((G,), jnp.int32),
            pltpu.VMEM((G,), jnp.int32),
            pltpu.VMEM((G, LANES), jnp.float32),
            pltpu.VMEM((G, LANES), jnp.float32),
            pltpu.VMEM((G, D), jnp.float32),
            pltpu.VMEM((G, D), jnp.float32),
            pltpu.VMEM((C, D), jnp.float32),
            pltpu.SemaphoreType.DMA,
            pltpu.SemaphoreType.DMA,
            pltpu.SemaphoreType.DMA,
            pltpu.SemaphoreType.DMA,
            pltpu.SemaphoreType.DMA,
            pltpu.SemaphoreType.DMA,
        ],
    )
    def sck(table_hbm, gidx_hbm, wb_hbm, out_hbm,
            idx0, idx1, w0, w1, rows0, rows1, out_v,
            si0, si1, sw0, sw1, sg0, sg1):
        wid = lax.axis_index("s") * NC + lax.axis_index("c")
        idx_v = (idx0, idx1)
        w_v = (w0, w1)
        rows_v = (rows0, rows1)
        si = (si0, si1)
        sw = (sw0, sw1)
        sg = (sg0, sg1)

        def start_idx(ch, slot):
            @pl.when(ch < NCHUNK)
            def _():
                ibase = (wid * PW + ch * C) * KNN
                pltpu.async_copy(gidx_hbm.at[pl.ds(ibase, G)], idx_v[slot],
                                 si[slot])

        def start_w(ch, slot):
            @pl.when(ch < NCHUNK)
            def _():
                ibase = (wid * PW + ch * C) * KNN
                pltpu.async_copy(wb_hbm.at[pl.ds(ibase, G)], w_v[slot],
                                 sw[slot])

        def wait_idx(slot):
            pltpu.make_async_copy(gidx_hbm.at[pl.ds(0, G)], idx_v[slot],
                                  si[slot]).wait()

        def start_gather(slot):
            pltpu.async_copy(table_hbm.at[idx_v[slot]], rows_v[slot], sg[slot])

        def wait_gather(slot):
            pltpu.make_async_copy(table_hbm.at[idx_v[slot]], rows_v[slot],
                                  sg[slot]).wait()
            pltpu.make_async_copy(wb_hbm.at[pl.ds(0, G)], w_v[slot],
                                  sw[slot]).wait()

        def compute(ch, slot):
            rv = rows_v[slot]
            wv = w_v[slot]

            @pl.loop(0, C)
            def _(r):
                a = wv[KNN * r]
                b = wv[KNN * r + 1]
                c_ = wv[KNN * r + 2]
                for c in range(D // LANES):
                    sl = pl.ds(c * LANES, LANES)
                    out_v[r, sl] = (rv[KNN * r, sl] * a +
                                    rv[KNN * r + 1, sl] * b +
                                    rv[KNN * r + 2, sl] * c_)

            pltpu.sync_copy(out_v, out_hbm.at[pl.ds(wid * PW + ch * C, C)])

        start_idx(0, 0)
        start_w(0, 0)
        wait_idx(0)
        start_gather(0)
        start_idx(1, 1)
        start_w(1, 1)

        @pl.loop(0, NCHUNK, step=2)
        def _(ch):
            wait_idx(1)
            start_gather(1)
            wait_gather(0)
            start_idx(ch + 2, 0)
            compute(ch, 0)
            start_w(ch + 2, 0)

            @pl.when(ch + 2 < NCHUNK)
            def _():
                wait_idx(0)
                start_gather(0)

            wait_gather(1)
            start_idx(ch + 3, 1)
            compute(ch + 1, 1)
            start_w(ch + 3, 1)

    return sck(table, gidx, wb)


@jax.jit
def kernel(xyz, sampled_xyz, features, sampled_features, masks):
    del features
    sxyz_t = sampled_xyz.transpose(0, 2, 1)
    mask_f = masks.astype(jnp.float32).reshape(B, N, 1)
    outs = []
    for b in range(B):
        gidx, wb = _tc_topk(xyz[b], sxyz_t[b], mask_f[b])
        out = _sc_interp(sampled_features[b],
                         gidx.reshape(N * KNN),
                         wb.reshape(N * KNN, LANES))
        outs.append(out)
    return jnp.stack(outs, axis=0)
